# Initial kernel scaffold; baseline (speedup 1.0000x reference)
import jax
import jax.numpy as jnp
from jax.experimental import pallas as pl


def kernel(x, edge_index, W1, b1, W2, b2):
    raise NotImplementedError("write your pallas kernel here")


# speedup vs baseline: 12.4345x; 12.4345x over previous
import functools

import jax
import jax.numpy as jnp
from jax import lax
from jax.experimental import pallas as pl
from jax.experimental.pallas import tpu as pltpu
from jax.experimental.pallas import tpu_sc as plsc

N = 10000
E = 320000
D = 128

NC = 2
NS = 16
NW = NC * NS
E_TILE = E // NW
CH = 128
N_FULL = E_TILE // CH
REM = E_TILE - N_FULL * CH
R_TILE = 640
R_LAST = N - (NS - 1) * R_TILE
DEG_W = 16

_sc_mesh = plsc.VectorSubcoreMesh(
    core_axis_name="c", subcore_axis_name="s", num_cores=NC, num_subcores=NS)


@functools.partial(
    pl.kernel,
    out_type=jax.ShapeDtypeStruct((NC, N, D), jnp.float32),
    mesh=_sc_mesh,
    scratch_types=[
        pltpu.VMEM((CH,), jnp.int32),
        pltpu.VMEM((CH,), jnp.int32),
        pltpu.VMEM((CH, D), jnp.float32),
        pltpu.VMEM((REM,), jnp.int32),
        pltpu.VMEM((REM,), jnp.int32),
        pltpu.VMEM((REM, D), jnp.float32),
        pltpu.VMEM_SHARED((N, D), jnp.float32),
        pltpu.SemaphoreType.DMA,
    ],
)
def _agg_kernel(src_hbm, dst_hbm, hs_hbm, zeros_hbm, out_hbm,
                idxs, idxd, rows, idxs_t, idxd_t, rows_t, acc, sem):
    c = lax.axis_index("c")
    s = lax.axis_index("s")
    wid = c * NS + s
    r0 = s * R_TILE

    def _init(nr):
        @pl.when(c == 0)
        def _():
            pltpu.sync_copy(hs_hbm.at[pl.ds(r0, nr)], acc.at[pl.ds(r0, nr)])

        @pl.when(c != 0)
        def _():
            pltpu.sync_copy(zeros_hbm.at[pl.ds(0, nr)], acc.at[pl.ds(r0, nr)])

    @pl.when(s < NS - 1)
    def _():
        _init(R_TILE)

    @pl.when(s == NS - 1)
    def _():
        _init(R_LAST)

    plsc.subcore_barrier()
    base = wid * E_TILE

    def body(j, carry):
        off = base + j * CH
        pltpu.sync_copy(src_hbm.at[pl.ds(off, CH)], idxs)
        pltpu.sync_copy(dst_hbm.at[pl.ds(off, CH)], idxd)
        pltpu.async_copy(hs_hbm.at[idxs], rows, sem).wait()
        pltpu.sync_copy(rows, acc.at[idxd], add=True)
        return carry

    lax.fori_loop(0, N_FULL, body, 0)
    off = base + N_FULL * CH
    pltpu.sync_copy(src_hbm.at[pl.ds(off, REM)], idxs_t)
    pltpu.sync_copy(dst_hbm.at[pl.ds(off, REM)], idxd_t)
    pltpu.async_copy(hs_hbm.at[idxs_t], rows_t, sem).wait()
    pltpu.sync_copy(rows_t, acc.at[idxd_t], add=True)
    plsc.subcore_barrier()

    @pl.when(s < NS - 1)
    def _():
        pltpu.sync_copy(acc.at[pl.ds(r0, R_TILE)],
                        out_hbm.at[c, pl.ds(r0, R_TILE)])

    @pl.when(s == NS - 1)
    def _():
        pltpu.sync_copy(acc.at[pl.ds(r0, R_LAST)],
                        out_hbm.at[c, pl.ds(r0, R_LAST)])


BR = 1000


def _disq(degp_ref):
    deg = degp_ref[0, :, 0:1] + degp_ref[1, :, 0:1]
    return lax.rsqrt(deg)


def _tc1_body(degp_ref, x_ref, w_ref, hs_ref):
    h = jnp.dot(x_ref[:], w_ref[:], preferred_element_type=jnp.float32)
    hs_ref[:] = h * _disq(degp_ref)


def _tc2_body(degp_ref, aggp_ref, b_ref, w_ref, hs_ref):
    dis = _disq(degp_ref)
    agg = aggp_ref[0] + aggp_ref[1]
    o1 = jnp.maximum(agg * dis + b_ref[:], 0.0)
    h = jnp.dot(o1, w_ref[:], preferred_element_type=jnp.float32)
    hs_ref[:] = h * dis


def _tc3_body(degp_ref, aggp_ref, b_ref, out_ref):
    agg = aggp_ref[0] + aggp_ref[1]
    out_ref[:] = agg * _disq(degp_ref) + b_ref[:]


_degp_spec = pl.BlockSpec((NC, BR, D), lambda i: (0, i, 0))
_aggp_spec = pl.BlockSpec((NC, BR, D), lambda i: (0, i, 0))
_row_spec = pl.BlockSpec((BR, D), lambda i: (i, 0))
_w_spec = pl.BlockSpec((D, D), lambda i: (0, 0))
_b_spec = pl.BlockSpec((1, D), lambda i: (0, 0))
_out_sds = jax.ShapeDtypeStruct((N, D), jnp.float32)

_tc1 = pl.pallas_call(
    _tc1_body, grid=(N // BR,),
    in_specs=[_degp_spec, _row_spec, _w_spec],
    out_specs=_row_spec, out_shape=_out_sds)

_tc2 = pl.pallas_call(
    _tc2_body, grid=(N // BR,),
    in_specs=[_degp_spec, _aggp_spec, _b_spec, _w_spec],
    out_specs=_row_spec, out_shape=_out_sds)

_tc3 = pl.pallas_call(
    _tc3_body, grid=(N // BR,),
    in_specs=[_degp_spec, _aggp_spec, _b_spec],
    out_specs=_row_spec, out_shape=_out_sds)


def kernel(x, edge_index, W1, b1, W2, b2):
    src = edge_index[0]
    dst = edge_index[1]
    ones_n = jnp.ones((N, D), jnp.float32)
    zeros_row = jnp.zeros((R_TILE, D), jnp.float32)

    degp = _agg_kernel(src, dst, ones_n, zeros_row)
    hs1 = _tc1(degp, x, W1)
    aggp1 = _agg_kernel(src, dst, hs1, zeros_row)
    hs2 = _tc2(degp, aggp1, b1.reshape(1, D), W2)
    aggp2 = _agg_kernel(src, dst, hs2, zeros_row)
    return _tc3(degp, aggp2, b2.reshape(1, D))
